# baseline (speedup 1.0000x reference)
import functools

import numpy as np

import jax
import jax.numpy as jnp
from jax import lax
from jax.experimental import pallas as pl
from jax.experimental.pallas import tpu as pltpu
from jax.experimental.pallas import tpu_sc as plsc

N_NODES = 10000
N_EDGES = 320000
D_IN = 128
D_OUT = 128
K = 4

NC = 2
NS = 16
NW = NC * NS
EW = N_EDGES // NW
G = 40
NCHUNK = EW // G
S = 10
NSUPER = NCHUNK // S
SG = S * G
ROW_SPLIT = 624
ROW_LAST = N_NODES - (NS - 1) * ROW_SPLIT

_HI = np.int32(-65536)


def _edge_mlp_body(a_ref, w1_ref, w2_ref, w3_ref, w4_ref,
                   o0_ref, o1_ref, o2_ref, o3_ref):
    a = a_ref[...]
    dn = (((1,), (1,)), ((), ()))
    lin = jnp.maximum(
        lax.dot_general(w1_ref[...], a, dn, preferred_element_type=jnp.float32),
        0.0)
    gat = jnp.tanh(lax.dot_general(w2_ref[...], a, dn,
                                   preferred_element_type=jnp.float32)) * \
          jnp.tanh(lax.dot_general(w3_ref[...], a, dn,
                                   preferred_element_type=jnp.float32))
    tmp = jnp.concatenate([lin, gat], axis=0)
    ea_t = jnp.maximum(
        jnp.dot(w4_ref[...], tmp, preferred_element_type=jnp.float32), 0.0)
    i = pl.program_id(0)
    o0_ref[pl.ds(i * _BE, _BE)] = ea_t[0]
    o1_ref[pl.ds(i * _BE, _BE)] = ea_t[1]
    o2_ref[pl.ds(i * _BE, _BE)] = ea_t[2]
    o3_ref[pl.ds(i * _BE, _BE)] = ea_t[3]


_BE = 16000


def _edge_mlp(edge_attr, w1, w2, w3, w4):
    grid = N_EDGES // _BE
    out1d = jax.ShapeDtypeStruct((N_EDGES,), jnp.float32)
    return pl.pallas_call(
        _edge_mlp_body,
        grid=(grid,),
        in_specs=[
            pl.BlockSpec((_BE, 16), lambda i: (i, 0)),
            pl.BlockSpec((32, 16), lambda i: (0, 0)),
            pl.BlockSpec((32, 16), lambda i: (0, 0)),
            pl.BlockSpec((32, 16), lambda i: (0, 0)),
            pl.BlockSpec((4, 64), lambda i: (0, 0)),
        ],
        out_specs=[pl.BlockSpec((N_EDGES,), lambda i: (0,))] * 4,
        out_shape=[out1d, out1d, out1d, out1d],
    )(edge_attr, w1, w2, w3, w4)


def _ypack_body(x_ref, w_ref, o_ref):
    xb = x_ref[...]
    m0 = jnp.dot(xb, w_ref[0], preferred_element_type=jnp.float32)
    m1 = jnp.dot(xb, w_ref[1], preferred_element_type=jnp.float32)
    u0 = lax.bitcast_convert_type(m0.astype(jnp.bfloat16),
                                  jnp.uint16).astype(jnp.int32)
    u1 = lax.bitcast_convert_type(m1.astype(jnp.bfloat16),
                                  jnp.uint16).astype(jnp.int32)
    o_ref[...] = lax.bitwise_or(u0, lax.shift_left(u1, 16))


def _ypack(x, wc):
    BN = 2000
    nb = N_NODES // BN
    return pl.pallas_call(
        _ypack_body,
        grid=(2, nb),
        in_specs=[
            pl.BlockSpec((BN, D_IN), lambda h, i: (i, 0)),
            pl.BlockSpec((2, D_IN, D_OUT), lambda h, i: (h, 0, 0)),
        ],
        out_specs=pl.BlockSpec((BN, D_OUT), lambda h, i: (h * (N_NODES // 2000) + i, 0)),
        out_shape=jax.ShapeDtypeStruct((2 * N_NODES, D_OUT), jnp.int32),
    )(x, wc)


_GATHER_DNUMS = lax.GatherDimensionNumbers(
    offset_dims=(), collapsed_slice_dims=(0,), start_index_map=(0,))


def _lane_bcast(w, lane):
    idx = jnp.full((16, 1), lane, jnp.int32)
    return lax.gather(w, idx, _GATHER_DNUMS, (1,),
                      mode=lax.GatherScatterMode.PROMISE_IN_BOUNDS)


def _sc_body(y2_hbm, src_hbm, dst_hbm, ea0_hbm, ea1_hbm, ea2_hbm, ea3_hbm,
             z_hbm, out_hbm,
             srcS, dstS, eaS, idx1b, dstb, rowsb, combb, acc,
             gsem0, gsem1, ssem0, ssem1, isem):
    c = lax.axis_index("c")
    s = lax.axis_index("s")
    wid = c * NS + s
    e0 = wid * EW
    gsems = (gsem0, gsem1)
    ssems = (ssem0, ssem1)
    ea_hbms = (ea0_hbm, ea1_hbm, ea2_hbm, ea3_hbm)

    @pl.when(s < NS - 1)
    def _():
        pltpu.sync_copy(z_hbm.at[pl.ds(0, ROW_SPLIT)],
                        acc.at[pl.ds(s * ROW_SPLIT, ROW_SPLIT)])

    @pl.when(s == NS - 1)
    def _():
        pltpu.sync_copy(z_hbm,
                        acc.at[pl.ds((NS - 1) * ROW_SPLIT, ROW_LAST)])

    plsc.subcore_barrier()

    def super_copies(sp, p):
        sbase = e0 + sp * SG
        yield src_hbm.at[pl.ds(sbase, SG)], srcS.at[pl.ds(p * SG, SG)]
        yield dst_hbm.at[pl.ds(sbase, SG)], dstS.at[pl.ds(p * SG, SG)]
        for i in range(K):
            yield (ea_hbms[i].at[pl.ds(sbase, SG)],
                   eaS.at[pl.ds((p * K + i) * SG, SG)])

    def super_load(sp, p):
        for a, v in super_copies(sp, p):
            pltpu.async_copy(a, v, isem)

    def super_wait(sp, p):
        for a, v in super_copies(sp, p):
            pltpu.make_async_copy(a, v, isem).wait()

    def start_gathers(p, k, b):
        for w0 in (0, 16, 24):
            v = srcS[pl.ds(p * SG + k * G + w0, 16)]
            idx1b[b, 0, pl.ds(w0, 16)] = v
            idx1b[b, 0, pl.ds(G + w0, 16)] = v + N_NODES
        pltpu.async_copy(y2_hbm.at[idx1b.at[b, 0]], rowsb.at[b], gsems[b])

    def wait_gathers(p, k, b):
        pltpu.make_async_copy(y2_hbm.at[idx1b.at[b, 0]],
                              rowsb.at[b], gsems[b]).wait()

    def drain_scatter(b):
        pltpu.make_async_copy(combb.at[b], acc.at[dstb.at[b, 0]],
                              ssems[b]).wait()

    def compute(p, k, b):
        def quad_body(q, carry):
            j0 = 4 * q
            woff = k * G + 16 * (q // 4)
            lane0 = 4 * (q - 4 * (q // 4))
            eb = p * (K * SG) + woff
            w0 = eaS[pl.ds(eb, 16)]
            w1 = eaS[pl.ds(eb + SG, 16)]
            w2 = eaS[pl.ds(eb + 2 * SG, 16)]
            w3 = eaS[pl.ds(eb + 3 * SG, 16)]
            for jj in range(4):
                j = j0 + jj
                lane = lane0 + jj
                bc0 = _lane_bcast(w0, lane)
                bc1 = _lane_bcast(w1, lane)
                bc2 = _lane_bcast(w2, lane)
                bc3 = _lane_bcast(w3, lane)
                for bk in range(8):
                    r0 = rowsb[b, j, pl.ds(bk * 16, 16)]
                    r1 = rowsb[b, G + j, pl.ds(bk * 16, 16)]
                    u00 = lax.bitcast_convert_type(lax.shift_left(r0, 16),
                                                   jnp.float32)
                    u01 = lax.bitcast_convert_type(lax.bitwise_and(r0, _HI),
                                                   jnp.float32)
                    u10 = lax.bitcast_convert_type(lax.shift_left(r1, 16),
                                                   jnp.float32)
                    u11 = lax.bitcast_convert_type(lax.bitwise_and(r1, _HI),
                                                   jnp.float32)
                    a = bc0 * u00 + bc1 * u01 + bc2 * u10 + bc3 * u11
                    combb[b, j, pl.ds(bk * 16, 16)] = a
            return carry

        lax.fori_loop(0, G // 4, quad_body, 0)

    def build_dstb(p, k, b):
        for w0 in (0, 16, 24):
            dstb[b, 0, pl.ds(w0, 16)] = dstS[pl.ds(p * SG + k * G + w0, 16)]

    def start_scatter(b):
        pltpu.async_copy(combb.at[b], acc.at[dstb.at[b, 0]], ssems[b], add=True)

    super_load(0, 0)
    super_wait(0, 0)
    start_gathers(0, 0, 0)

    def super_body(sp, carry):
        p = lax.rem(sp, 2)
        pn = lax.rem(sp + 1, 2)

        @pl.when(sp >= 1)
        def _():
            drain_scatter(0)
            drain_scatter(1)

        @pl.when(sp + 1 < NSUPER)
        def _():
            super_load(sp + 1, pn)

        def pair_body(i2, carry2):
            k0 = 2 * i2
            wait_gathers(p, k0, 0)
            start_gathers(p, k0 + 1, 1)

            @pl.when(i2 >= 1)
            def _():
                drain_scatter(0)

            compute(p, k0, 0)
            build_dstb(p, k0, 0)
            start_scatter(0)

            wait_gathers(p, k0 + 1, 1)

            @pl.when(i2 + 1 < S // 2)
            def _():
                start_gathers(p, k0 + 2, 0)

            @pl.when(jnp.logical_and(i2 + 1 == S // 2, sp + 1 < NSUPER))
            def _():
                super_wait(sp + 1, pn)
                start_gathers(pn, 0, 0)

            @pl.when(i2 >= 1)
            def _():
                drain_scatter(1)

            compute(p, k0 + 1, 1)
            build_dstb(p, k0 + 1, 1)
            start_scatter(1)
            return carry2

        lax.fori_loop(0, S // 2, pair_body, 0)
        return carry

    lax.fori_loop(0, NSUPER, super_body, 0)
    drain_scatter(0)
    drain_scatter(1)

    plsc.subcore_barrier()

    @pl.when(s < NS - 1)
    def _():
        r0 = s * ROW_SPLIT
        pltpu.sync_copy(acc.at[pl.ds(r0, ROW_SPLIT)],
                        out_hbm.at[c, pl.ds(r0, ROW_SPLIT)])

    @pl.when(s == NS - 1)
    def _():
        r0 = (NS - 1) * ROW_SPLIT
        pltpu.sync_copy(acc.at[pl.ds(r0, ROW_LAST)],
                        out_hbm.at[c, pl.ds(r0, ROW_LAST)])


def _sc_scatter(y2, src, dst, ea0, ea1, ea2, ea3, zeros):
    mesh = plsc.VectorSubcoreMesh(core_axis_name="c", subcore_axis_name="s")
    f = functools.partial(
        pl.kernel,
        out_type=jax.ShapeDtypeStruct((NC, N_NODES, D_OUT), jnp.float32),
        mesh=mesh,
        scratch_types=[
            pltpu.VMEM((2 * SG,), jnp.int32),
            pltpu.VMEM((2 * SG,), jnp.int32),
            pltpu.VMEM((2 * K * SG,), jnp.float32),
            pltpu.VMEM((2, 1, 2 * G), jnp.int32),
            pltpu.VMEM((2, 1, G), jnp.int32),
            pltpu.VMEM((2, 2 * G, D_OUT), jnp.int32),
            pltpu.VMEM((2, G, D_OUT), jnp.float32),
            pltpu.VMEM_SHARED((N_NODES, D_OUT), jnp.float32),
            pltpu.SemaphoreType.DMA,
            pltpu.SemaphoreType.DMA,
            pltpu.SemaphoreType.DMA,
            pltpu.SemaphoreType.DMA,
            pltpu.SemaphoreType.DMA,
        ],
    )(_sc_body)
    return f(y2, src, dst, ea0, ea1, ea2, ea3, zeros)


ROWC = 312
ROWC_LAST = N_NODES - (NW - 1) * ROWC


def _sc_combine_body(p_hbm, bias_hbm, out_hbm, b0, b1, bias_v):
    c = lax.axis_index("c")
    s = lax.axis_index("s")
    wid = c * NS + s
    r0 = wid * ROWC
    pltpu.sync_copy(bias_hbm, bias_v)

    def do(sz, b0v, b1v):
        pltpu.sync_copy(p_hbm.at[0, pl.ds(r0, sz)], b0v)
        pltpu.sync_copy(p_hbm.at[1, pl.ds(r0, sz)], b1v)

        def row_body(r, carry):
            for w in range(8):
                cs = pl.ds(w * 16, 16)
                b0v[r, cs] = jnp.maximum(
                    b0v[r, cs] + b1v[r, cs] + bias_v[cs], 0.0)
            return carry

        lax.fori_loop(0, sz, row_body, 0)
        pltpu.sync_copy(b0v, out_hbm.at[pl.ds(r0, sz)])

    @pl.when(wid < NW - 1)
    def _():
        do(ROWC, b0.at[pl.ds(0, ROWC)], b1.at[pl.ds(0, ROWC)])

    @pl.when(wid == NW - 1)
    def _():
        do(ROWC_LAST, b0, b1)


def _sc_combine(partials, bias):
    mesh = plsc.VectorSubcoreMesh(core_axis_name="c", subcore_axis_name="s")
    f = functools.partial(
        pl.kernel,
        out_type=jax.ShapeDtypeStruct((N_NODES, D_OUT), jnp.float32),
        mesh=mesh,
        scratch_types=[
            pltpu.VMEM((ROWC_LAST, D_OUT), jnp.float32),
            pltpu.VMEM((ROWC_LAST, D_OUT), jnp.float32),
            pltpu.VMEM((D_OUT,), jnp.float32),
        ],
    )(_sc_combine_body)
    return f(partials, bias)


def kernel(x, edge_index, edge_attr, W1, W2, W3, W4, Wc, bias):
    src = edge_index[0].astype(jnp.int32)
    dst = edge_index[1].astype(jnp.int32)
    ea0, ea1, ea2, ea3 = _edge_mlp(edge_attr, W1, W2, W3, W4)
    y2 = _ypack(x, Wc)
    zeros = jnp.zeros((ROW_LAST, D_OUT), jnp.float32)
    partials = _sc_scatter(y2, src, dst, ea0, ea1, ea2, ea3, zeros)
    return _sc_combine(partials, bias)


# speedup vs baseline: 1.1747x; 1.1747x over previous
import functools

import numpy as np

import jax
import jax.numpy as jnp
from jax import lax
from jax.experimental import pallas as pl
from jax.experimental.pallas import tpu as pltpu
from jax.experimental.pallas import tpu_sc as plsc

N_NODES = 10000
N_EDGES = 320000
D_IN = 128
D_OUT = 128
K = 4

NC = 2
NS = 16
NW = NC * NS
EW = N_EDGES // NW
G = 40
NCHUNK = EW // G
S = 10
NSUPER = NCHUNK // S
SG = S * G
ROW_SPLIT = 624
ROW_LAST = N_NODES - (NS - 1) * ROW_SPLIT

_HI = np.int32(-65536)


def _edge_mlp_body(a_ref, w1_ref, w2_ref, w3_ref, w4_ref,
                   o0_ref, o1_ref, o2_ref, o3_ref):
    a = a_ref[...]
    lin = jnp.maximum(
        jnp.dot(w1_ref[...], a, preferred_element_type=jnp.float32),
        0.0)
    gat = jnp.tanh(jnp.dot(w2_ref[...], a,
                           preferred_element_type=jnp.float32)) * \
          jnp.tanh(jnp.dot(w3_ref[...], a,
                           preferred_element_type=jnp.float32))
    tmp = jnp.concatenate([lin, gat], axis=0)
    ea_t = jnp.maximum(
        jnp.dot(w4_ref[...], tmp, preferred_element_type=jnp.float32), 0.0)
    i = pl.program_id(0)
    o0_ref[pl.ds(i * _BE, _BE)] = ea_t[0]
    o1_ref[pl.ds(i * _BE, _BE)] = ea_t[1]
    o2_ref[pl.ds(i * _BE, _BE)] = ea_t[2]
    o3_ref[pl.ds(i * _BE, _BE)] = ea_t[3]


_BE = 16000


def _edge_mlp(edge_attr_t, w1, w2, w3, w4):
    grid = N_EDGES // _BE
    out1d = jax.ShapeDtypeStruct((N_EDGES,), jnp.float32)
    return pl.pallas_call(
        _edge_mlp_body,
        grid=(grid,),
        in_specs=[
            pl.BlockSpec((16, _BE), lambda i: (0, i)),
            pl.BlockSpec((32, 16), lambda i: (0, 0)),
            pl.BlockSpec((32, 16), lambda i: (0, 0)),
            pl.BlockSpec((32, 16), lambda i: (0, 0)),
            pl.BlockSpec((4, 64), lambda i: (0, 0)),
        ],
        out_specs=[pl.BlockSpec((N_EDGES,), lambda i: (0,))] * 4,
        out_shape=[out1d, out1d, out1d, out1d],
    )(edge_attr_t, w1, w2, w3, w4)


def _ypack_body(x_ref, w_ref, o_ref):
    xb = x_ref[...]
    m0 = jnp.dot(xb, w_ref[0], preferred_element_type=jnp.float32)
    m1 = jnp.dot(xb, w_ref[1], preferred_element_type=jnp.float32)
    u0 = lax.bitcast_convert_type(m0.astype(jnp.bfloat16),
                                  jnp.uint16).astype(jnp.int32)
    u1 = lax.bitcast_convert_type(m1.astype(jnp.bfloat16),
                                  jnp.uint16).astype(jnp.int32)
    o_ref[...] = lax.bitwise_or(u0, lax.shift_left(u1, 16))


def _ypack(x, wc):
    BN = 2000
    nb = N_NODES // BN
    return pl.pallas_call(
        _ypack_body,
        grid=(2, nb),
        in_specs=[
            pl.BlockSpec((BN, D_IN), lambda h, i: (i, 0)),
            pl.BlockSpec((2, D_IN, D_OUT), lambda h, i: (h, 0, 0)),
        ],
        out_specs=pl.BlockSpec((BN, D_OUT), lambda h, i: (h * (N_NODES // 2000) + i, 0)),
        out_shape=jax.ShapeDtypeStruct((2 * N_NODES, D_OUT), jnp.int32),
    )(x, wc)


_GATHER_DNUMS = lax.GatherDimensionNumbers(
    offset_dims=(), collapsed_slice_dims=(0,), start_index_map=(0,))


def _lane_bcast(w, lane):
    idx = jnp.full((16, 1), lane, jnp.int32)
    return lax.gather(w, idx, _GATHER_DNUMS, (1,),
                      mode=lax.GatherScatterMode.PROMISE_IN_BOUNDS)


def _sc_body(y2_hbm, src_hbm, dst_hbm, ea0_hbm, ea1_hbm, ea2_hbm, ea3_hbm,
             z_hbm, out_hbm,
             srcS, dstS, eaS, idx1b, dstb, rowsb, combb, acc,
             gsem0, gsem1, ssem0, ssem1, isem):
    c = lax.axis_index("c")
    s = lax.axis_index("s")
    wid = c * NS + s
    e0 = wid * EW
    gsems = (gsem0, gsem1)
    ssems = (ssem0, ssem1)
    ea_hbms = (ea0_hbm, ea1_hbm, ea2_hbm, ea3_hbm)

    @pl.when(s < NS - 1)
    def _():
        pltpu.sync_copy(z_hbm.at[pl.ds(0, ROW_SPLIT)],
                        acc.at[pl.ds(s * ROW_SPLIT, ROW_SPLIT)])

    @pl.when(s == NS - 1)
    def _():
        pltpu.sync_copy(z_hbm,
                        acc.at[pl.ds((NS - 1) * ROW_SPLIT, ROW_LAST)])

    plsc.subcore_barrier()

    def super_copies(sp, p):
        sbase = e0 + sp * SG
        yield src_hbm.at[pl.ds(sbase, SG)], srcS.at[pl.ds(p * SG, SG)]
        yield dst_hbm.at[pl.ds(sbase, SG)], dstS.at[pl.ds(p * SG, SG)]
        for i in range(K):
            yield (ea_hbms[i].at[pl.ds(sbase, SG)],
                   eaS.at[pl.ds((p * K + i) * SG, SG)])

    def super_load(sp, p):
        for a, v in super_copies(sp, p):
            pltpu.async_copy(a, v, isem)

    def super_wait(sp, p):
        for a, v in super_copies(sp, p):
            pltpu.make_async_copy(a, v, isem).wait()

    def start_gathers(p, k, b):
        for w0 in (0, 16, 24):
            v = srcS[pl.ds(p * SG + k * G + w0, 16)]
            idx1b[b, 0, pl.ds(w0, 16)] = v
            idx1b[b, 0, pl.ds(G + w0, 16)] = v + N_NODES
        pltpu.async_copy(y2_hbm.at[idx1b.at[b, 0]], rowsb.at[b], gsems[b])

    def wait_gathers(p, k, b):
        pltpu.make_async_copy(y2_hbm.at[idx1b.at[b, 0]],
                              rowsb.at[b], gsems[b]).wait()

    def drain_scatter(b):
        pltpu.make_async_copy(combb.at[b], acc.at[dstb.at[b, 0]],
                              ssems[b]).wait()

    def compute(p, k, b):
        def quad_body(q, carry):
            j0 = 4 * q
            woff = k * G + 16 * (q // 4)
            lane0 = 4 * (q - 4 * (q // 4))
            eb = p * (K * SG) + woff
            w0 = eaS[pl.ds(eb, 16)]
            w1 = eaS[pl.ds(eb + SG, 16)]
            w2 = eaS[pl.ds(eb + 2 * SG, 16)]
            w3 = eaS[pl.ds(eb + 3 * SG, 16)]
            for jj in range(4):
                j = j0 + jj
                lane = lane0 + jj
                bc0 = _lane_bcast(w0, lane)
                bc1 = _lane_bcast(w1, lane)
                bc2 = _lane_bcast(w2, lane)
                bc3 = _lane_bcast(w3, lane)
                for bk in range(8):
                    r0 = rowsb[b, j, pl.ds(bk * 16, 16)]
                    r1 = rowsb[b, G + j, pl.ds(bk * 16, 16)]
                    u00 = lax.bitcast_convert_type(lax.shift_left(r0, 16),
                                                   jnp.float32)
                    u01 = lax.bitcast_convert_type(lax.bitwise_and(r0, _HI),
                                                   jnp.float32)
                    u10 = lax.bitcast_convert_type(lax.shift_left(r1, 16),
                                                   jnp.float32)
                    u11 = lax.bitcast_convert_type(lax.bitwise_and(r1, _HI),
                                                   jnp.float32)
                    a = bc0 * u00 + bc1 * u01 + bc2 * u10 + bc3 * u11
                    combb[b, j, pl.ds(bk * 16, 16)] = a
            return carry

        lax.fori_loop(0, G // 4, quad_body, 0)

    def build_dstb(p, k, b):
        for w0 in (0, 16, 24):
            dstb[b, 0, pl.ds(w0, 16)] = dstS[pl.ds(p * SG + k * G + w0, 16)]

    def start_scatter(b):
        pltpu.async_copy(combb.at[b], acc.at[dstb.at[b, 0]], ssems[b], add=True)

    super_load(0, 0)
    super_wait(0, 0)
    start_gathers(0, 0, 0)

    def super_body(sp, carry):
        p = lax.rem(sp, 2)
        pn = lax.rem(sp + 1, 2)

        @pl.when(sp >= 1)
        def _():
            drain_scatter(0)
            drain_scatter(1)

        @pl.when(sp + 1 < NSUPER)
        def _():
            super_load(sp + 1, pn)

        def pair_body(i2, carry2):
            k0 = 2 * i2
            wait_gathers(p, k0, 0)
            start_gathers(p, k0 + 1, 1)

            @pl.when(i2 >= 1)
            def _():
                drain_scatter(0)

            compute(p, k0, 0)
            build_dstb(p, k0, 0)
            start_scatter(0)

            wait_gathers(p, k0 + 1, 1)

            @pl.when(i2 + 1 < S // 2)
            def _():
                start_gathers(p, k0 + 2, 0)

            @pl.when(jnp.logical_and(i2 + 1 == S // 2, sp + 1 < NSUPER))
            def _():
                super_wait(sp + 1, pn)
                start_gathers(pn, 0, 0)

            @pl.when(i2 >= 1)
            def _():
                drain_scatter(1)

            compute(p, k0 + 1, 1)
            build_dstb(p, k0 + 1, 1)
            start_scatter(1)
            return carry2

        lax.fori_loop(0, S // 2, pair_body, 0)
        return carry

    lax.fori_loop(0, NSUPER, super_body, 0)
    drain_scatter(0)
    drain_scatter(1)

    plsc.subcore_barrier()

    @pl.when(s < NS - 1)
    def _():
        r0 = s * ROW_SPLIT
        pltpu.sync_copy(acc.at[pl.ds(r0, ROW_SPLIT)],
                        out_hbm.at[c, pl.ds(r0, ROW_SPLIT)])

    @pl.when(s == NS - 1)
    def _():
        r0 = (NS - 1) * ROW_SPLIT
        pltpu.sync_copy(acc.at[pl.ds(r0, ROW_LAST)],
                        out_hbm.at[c, pl.ds(r0, ROW_LAST)])


def _sc_scatter(y2, src, dst, ea0, ea1, ea2, ea3, zeros):
    mesh = plsc.VectorSubcoreMesh(core_axis_name="c", subcore_axis_name="s")
    f = functools.partial(
        pl.kernel,
        out_type=jax.ShapeDtypeStruct((NC, N_NODES, D_OUT), jnp.float32),
        mesh=mesh,
        scratch_types=[
            pltpu.VMEM((2 * SG,), jnp.int32),
            pltpu.VMEM((2 * SG,), jnp.int32),
            pltpu.VMEM((2 * K * SG,), jnp.float32),
            pltpu.VMEM((2, 1, 2 * G), jnp.int32),
            pltpu.VMEM((2, 1, G), jnp.int32),
            pltpu.VMEM((2, 2 * G, D_OUT), jnp.int32),
            pltpu.VMEM((2, G, D_OUT), jnp.float32),
            pltpu.VMEM_SHARED((N_NODES, D_OUT), jnp.float32),
            pltpu.SemaphoreType.DMA,
            pltpu.SemaphoreType.DMA,
            pltpu.SemaphoreType.DMA,
            pltpu.SemaphoreType.DMA,
            pltpu.SemaphoreType.DMA,
        ],
    )(_sc_body)
    return f(y2, src, dst, ea0, ea1, ea2, ea3, zeros)


ROWC = 312
ROWC_LAST = N_NODES - (NW - 1) * ROWC


def _sc_combine_body(p_hbm, bias_hbm, out_hbm, b0, b1, bias_v):
    c = lax.axis_index("c")
    s = lax.axis_index("s")
    wid = c * NS + s
    r0 = wid * ROWC
    pltpu.sync_copy(bias_hbm, bias_v)

    def do(sz, b0v, b1v):
        pltpu.sync_copy(p_hbm.at[0, pl.ds(r0, sz)], b0v)
        pltpu.sync_copy(p_hbm.at[1, pl.ds(r0, sz)], b1v)

        def row_body(r, carry):
            for w in range(8):
                cs = pl.ds(w * 16, 16)
                b0v[r, cs] = jnp.maximum(
                    b0v[r, cs] + b1v[r, cs] + bias_v[cs], 0.0)
            return carry

        lax.fori_loop(0, sz, row_body, 0)
        pltpu.sync_copy(b0v, out_hbm.at[pl.ds(r0, sz)])

    @pl.when(wid < NW - 1)
    def _():
        do(ROWC, b0.at[pl.ds(0, ROWC)], b1.at[pl.ds(0, ROWC)])

    @pl.when(wid == NW - 1)
    def _():
        do(ROWC_LAST, b0, b1)


def _sc_combine(partials, bias):
    mesh = plsc.VectorSubcoreMesh(core_axis_name="c", subcore_axis_name="s")
    f = functools.partial(
        pl.kernel,
        out_type=jax.ShapeDtypeStruct((N_NODES, D_OUT), jnp.float32),
        mesh=mesh,
        scratch_types=[
            pltpu.VMEM((ROWC_LAST, D_OUT), jnp.float32),
            pltpu.VMEM((ROWC_LAST, D_OUT), jnp.float32),
            pltpu.VMEM((D_OUT,), jnp.float32),
        ],
    )(_sc_combine_body)
    return f(partials, bias)


def kernel(x, edge_index, edge_attr, W1, W2, W3, W4, Wc, bias):
    src = edge_index[0].astype(jnp.int32)
    dst = edge_index[1].astype(jnp.int32)
    ea0, ea1, ea2, ea3 = _edge_mlp(edge_attr.T, W1, W2, W3, W4)
    y2 = _ypack(x, Wc)
    zeros = jnp.zeros((ROW_LAST, D_OUT), jnp.float32)
    partials = _sc_scatter(y2, src, dst, ea0, ea1, ea2, ea3, zeros)
    return _sc_combine(partials, bias)
